# baseline (speedup 1.0000x reference)
import functools

import jax
import jax.numpy as jnp
from jax import lax
from jax.experimental import pallas as pl
from jax.experimental.pallas import tpu as pltpu
from jax.experimental.pallas import tpu_sc as plsc

_N, _K, _D = 10000, 32, 128
_ALPHA = 0.1
_LANES = 16
_CREG = _D // _LANES
_B = 8
_RPB = _B * _K
_IDX_CHUNK = 128
_NCHUNK = _RPB // _IDX_CHUNK
_NBLK = _N // _B
_NW = 32
_BASE = _NBLK // _NW
_EXTRA = _NBLK % _NW
_NBUF = 3


def _sc_body(x_hbm, idx_hbm, h_hbm, out_hbm,
             idx_v, rows_v, h_v, out_v,
             sem_g0, sem_g1, sem_g2, sem_h0, sem_h1, sem_h2):
    wid = lax.axis_index("s") * 2 + lax.axis_index("c")
    nb = _BASE + jnp.where(wid < _EXTRA, 1, 0)
    sb = wid * _BASE + jnp.minimum(wid, _EXTRA)
    end = sb + nb

    sem_g = (sem_g0, sem_g1, sem_g2)
    sem_h = (sem_h0, sem_h1, sem_h2)

    def issue(g, slot):
        pltpu.sync_copy(idx_hbm.at[pl.ds(g * _NCHUNK, _NCHUNK)], idx_v.at[slot])
        pltpu.async_copy(h_hbm.at[pl.ds(g * _B, _B)], h_v.at[slot], sem_h[slot])
        for j in range(_NCHUNK):
            pltpu.async_copy(
                x_hbm.at[idx_v.at[slot, j]],
                rows_v.at[slot, pl.ds(j * _IDX_CHUNK, _IDX_CHUNK)],
                sem_g[slot])

    def drain(g, slot):
        for j in range(_NCHUNK):
            pltpu.make_async_copy(
                x_hbm.at[idx_v.at[slot, j]],
                rows_v.at[slot, pl.ds(j * _IDX_CHUNK, _IDX_CHUNK)],
                sem_g[slot]).wait()
        pltpu.make_async_copy(
            h_hbm.at[pl.ds(g * _B, _B)], h_v.at[slot], sem_h[slot]).wait()

    def compute(g, slot):
        for b in range(_B):
            base_r = b * _K
            acc0 = tuple(rows_v[slot, base_r, pl.ds(c * _LANES, _LANES)]
                         for c in range(_CREG))

            def body(k, acc, base_r=base_r, slot=slot):
                return tuple(
                    acc[c] + rows_v[slot, base_r + k, pl.ds(c * _LANES, _LANES)]
                    for c in range(_CREG))

            acc = lax.fori_loop(1, _K, body, acc0)
            for c in range(_CREG):
                out_v[b, pl.ds(c * _LANES, _LANES)] = (
                    (1.0 - _ALPHA) * acc[c]
                    + _ALPHA * h_v[slot, b, pl.ds(c * _LANES, _LANES)])
        pltpu.sync_copy(out_v, out_hbm.at[pl.ds(g * _B, _B)])

    for slot in range(_NBUF):
        @pl.when(slot < nb)
        def _(slot=slot):
            issue(sb + slot, slot)

    def outer(i, carry):
        for slot in range(_NBUF):
            g = sb + _NBUF * i + slot

            @pl.when(g < end)
            def _(g=g, slot=slot):
                drain(g, slot)
                compute(g, slot)

                @pl.when(g + _NBUF < end)
                def _(g=g, slot=slot):
                    issue(g + _NBUF, slot)

        return carry

    lax.fori_loop(0, (nb + _NBUF - 1) // _NBUF, outer, 0)


_sc_call = functools.partial(
    pl.kernel,
    out_type=jax.ShapeDtypeStruct((_N, _D), jnp.float32),
    mesh=plsc.VectorSubcoreMesh(core_axis_name="c", subcore_axis_name="s"),
    compiler_params=pltpu.CompilerParams(skip_device_barrier=True),
    scratch_types=[
        pltpu.VMEM((_NBUF, _NCHUNK, _IDX_CHUNK), jnp.int32),
        pltpu.VMEM((_NBUF, _RPB, _D), jnp.float32),
        pltpu.VMEM((_NBUF, _B, _D), jnp.float32),
        pltpu.VMEM((_B, _D), jnp.float32),
        pltpu.SemaphoreType.DMA,
        pltpu.SemaphoreType.DMA,
        pltpu.SemaphoreType.DMA,
        pltpu.SemaphoreType.DMA,
        pltpu.SemaphoreType.DMA,
        pltpu.SemaphoreType.DMA,
    ],
)(_sc_body)


def kernel(x, neighbor_agg, h, neighbor):
    idx2d = neighbor_agg.astype(jnp.int32).reshape(_N * _K // _IDX_CHUNK,
                                                   _IDX_CHUNK)
    out = _sc_call(x, idx2d, h)
    return (out, neighbor)
